# baseline (speedup 1.0000x reference)
import functools

import jax
import jax.numpy as jnp
from jax import lax
from jax.experimental import pallas as pl
from jax.experimental.pallas import tpu as pltpu
from jax.experimental.pallas import tpu_sc as plsc

_B = 10000
_S2 = 10
_S1 = 50
_F = 3
_N = 100000
_EMB = 128
_BPAD = 10240
_BLK = 512
_G = _B // 16
_WPC = (11, 11, 10)
_GPW = (57, 57, 63)


def _sc_colsum(features, neigh):
    mesh = plsc.VectorSubcoreMesh(core_axis_name="c", subcore_axis_name="s")
    _EXR = 400
    _EXT = _N // _EXR

    @functools.partial(
        pl.kernel,
        out_type=jax.ShapeDtypeStruct((_S2 * _F * _BPAD,), jnp.float32),
        mesh=mesh,
        scratch_types=[
            pltpu.VMEM((_N,), jnp.float32),
            pltpu.VMEM((_S2 * 1008,), jnp.float32),
            pltpu.SemaphoreType.DMA,
            pltpu.SemaphoreType.DMA,
        ],
        compiler_params=pltpu.CompilerParams(use_tc_tiling_on_sc=False,
                                             needs_layout_passes=False),
    )
    def k(feat_hbm, neigh_hbm, out_hbm, col_v, stage_v, sem_a, sem_b):
        wid = lax.axis_index("s") * 2 + lax.axis_index("c")
        iota16 = lax.iota(jnp.int32, 16)

        def ex_start(buf, sem, t):
            pltpu.async_copy(feat_hbm.at[pl.ds(t * _EXR, _EXR), :], buf, sem)

        def ex_wait(buf, sem):
            pltpu.make_async_copy(feat_hbm.at[pl.ds(0, _EXR), :], buf,
                                  sem).wait()

        def ex_compute(buf, t, c):
            cvec = jnp.full((16,), c, jnp.int32)
            for s in range(_EXR // 16):
                col_v[pl.ds(t * _EXR + s * 16, 16)] = plsc.load_gather(
                    buf, [iota16 + s * 16, cvec])

        def extraction(c):
            def scoped(ex_a, ex_b):
                ex_start(ex_a, sem_a, 0)
                ex_start(ex_b, sem_b, 1)

                def exbody(tt, carry):
                    t0 = tt * 2
                    ex_wait(ex_a, sem_a)
                    ex_compute(ex_a, t0, c)

                    @pl.when(tt < _EXT // 2 - 1)
                    def _():
                        ex_start(ex_a, sem_a, t0 + 2)

                    ex_wait(ex_b, sem_b)
                    ex_compute(ex_b, t0 + 1, c)

                    @pl.when(tt < _EXT // 2 - 1)
                    def _():
                        ex_start(ex_b, sem_b, t0 + 3)

                    return carry

                lax.fori_loop(0, _EXT // 2, exbody, 0)

            pl.run_scoped(scoped,
                          pltpu.VMEM((_EXR, _F), jnp.float32),
                          pltpu.VMEM((_EXR, _F), jnp.float32))

        def g_start(buf, sem, gidx):
            pltpu.async_copy(neigh_hbm.at[pl.ds(gidx * 16, 16), :, :], buf,
                             sem)

        def g_wait(buf, sem):
            pltpu.make_async_copy(neigh_hbm.at[pl.ds(0, 16), :, :], buf,
                                  sem).wait()

        def g_compute(buf, g):
            def n_body(n, carry):
                nvec = jnp.full((16,), n, jnp.int32)
                acc = jnp.zeros((16,), jnp.float32)
                for j in range(_S1):
                    iv = plsc.load_gather(
                        buf, [iota16, nvec, jnp.full((16,), j, jnp.int32)])
                    acc = acc + plsc.load_gather(col_v, [iv])
                stage_v[pl.ds(n * 1008 + g * 16, 16)] = acc
                return carry

            lax.fori_loop(0, _S2, n_body, 0)

        def run_col(c, lo, gpw):
            r = wid - lo
            extraction(c)
            half = (gpw + 1) // 2

            def gclamp(g):
                return jnp.minimum(r * gpw + jnp.minimum(g, gpw - 1), _G - 1)

            def scoped(buf_a, buf_b):
                g_start(buf_a, sem_a, gclamp(0))
                g_start(buf_b, sem_b, gclamp(1))

                def gbody(gg, carry):
                    g0 = gg * 2
                    g_wait(buf_a, sem_a)
                    g_compute(buf_a, jnp.minimum(g0, gpw - 1))

                    @pl.when(gg < half - 1)
                    def _():
                        g_start(buf_a, sem_a, gclamp(g0 + 2))

                    g_wait(buf_b, sem_b)
                    g_compute(buf_b, jnp.minimum(g0 + 1, gpw - 1))

                    @pl.when(gg < half - 1)
                    def _():
                        g_start(buf_b, sem_b, gclamp(g0 + 3))

                    return carry

                lax.fori_loop(0, half, gbody, 0)

            pl.run_scoped(scoped,
                          pltpu.VMEM((16, _S2, _S1), jnp.int32),
                          pltpu.VMEM((16, _S2, _S1), jnp.int32))
            sw = gpw * 16
            for n in range(_S2):
                pltpu.sync_copy(
                    stage_v.at[pl.ds(n * 1008, sw)],
                    out_hbm.at[pl.ds((n * _F + c) * _BPAD + r * sw, sw)])

        lo = 0
        for c in range(3):
            hi = lo + _WPC[c]

            @pl.when(jnp.logical_and(wid >= lo, wid < hi))
            def _(c=c, lo=lo):
                run_col(c, lo, _GPW[c])

            lo = hi

    return k(features, neigh)


def _bf(x):
    return x.astype(jnp.bfloat16)


def _dot(a, b):
    return jnp.dot(a, _bf(b), preferred_element_type=jnp.float32)


def _scores(m, w1b, w2b, l1b, b1v, l2b, b2, l3b, b3):
    nblk = _BPAD // _BLK

    def body(m_ref, w1_ref, w2_ref, l1_ref, b1_ref, l2_ref, b2_ref,
             l3_ref, b3_ref, o_ref):
        w1 = w1_ref[...]
        acc = jnp.zeros((_EMB, _BLK), jnp.float32)
        for n in range(_S2):
            acc = acc + jnp.maximum(_dot(w1, m_ref[n] / 50.0), 0.0)
        h2 = jnp.maximum(_dot(w2_ref[...], acc / 10.0), 0.0)
        x1 = jnp.maximum(_dot(l1_ref[...], h2) + b1_ref[...], 0.0)
        x2 = jnp.maximum(_dot(l2_ref[...], x1) + b2_ref[...], 0.0)
        s = _dot(l3_ref[...], x2) + b3_ref[...]
        i = pl.program_id(0)
        colid = i * _BLK + lax.broadcasted_iota(jnp.int32, (1, _BLK), 1)
        o_ref[...] = jnp.where(colid < _B, s, -1e30)

    return pl.pallas_call(
        body,
        grid=(nblk,),
        in_specs=[
            pl.BlockSpec((_S2, _F, _BLK), lambda i: (0, 0, i)),
            pl.BlockSpec((_EMB, _F), lambda i: (0, 0)),
            pl.BlockSpec((_EMB, _EMB), lambda i: (0, 0)),
            pl.BlockSpec((_EMB, _EMB), lambda i: (0, 0)),
            pl.BlockSpec((_EMB, 1), lambda i: (0, 0)),
            pl.BlockSpec((64, _EMB), lambda i: (0, 0)),
            pl.BlockSpec((64, 1), lambda i: (0, 0)),
            pl.BlockSpec((1, 64), lambda i: (0, 0)),
            pl.BlockSpec((1, 1), lambda i: (0, 0)),
        ],
        out_specs=pl.BlockSpec((1, _BLK), lambda i: (0, i)),
        out_shape=jax.ShapeDtypeStruct((1, _BPAD), jnp.float32),
    )(m, w1b, w2b, l1b, b1v, l2b, b2, l3b, b3)


def _softmax(s):
    def body(s_ref, o_ref):
        x = s_ref[...]
        e = jnp.exp(x - jnp.max(x))
        o_ref[...] = (e / jnp.sum(e))[:, :_B]

    return pl.pallas_call(
        body, out_shape=jax.ShapeDtypeStruct((1, _B), jnp.float32))(s)


def kernel(nodes, features, partition_size, partition_edges, neigh2, neigh1,
           W1, W2, L1_w, L1_b, L2_w, L2_b, L3_w, L3_b):
    m = _sc_colsum(features, neigh1.astype(jnp.int32)).reshape(
        _S2, _F, _BPAD)

    rem = (jnp.asarray(partition_size, jnp.float32)
           - jnp.float32(partition_edges.shape[0]))
    rem_b = rem.astype(jnp.bfloat16).astype(jnp.float32)
    c128 = L1_w[:, _EMB].astype(jnp.bfloat16).astype(jnp.float32)
    b1v = (L1_b + c128 * rem_b).reshape(_EMB, 1)
    w1b = W1.astype(jnp.bfloat16)
    w2b = W2.astype(jnp.bfloat16)
    l1b = L1_w[:, :_EMB].astype(jnp.bfloat16)
    l2b = L2_w.astype(jnp.bfloat16)
    l3b = L3_w.astype(jnp.bfloat16)
    b2 = L2_b.reshape(64, 1)
    b3 = L3_b.reshape(1, 1)

    s = _scores(m, w1b, w2b, l1b, b1v, l2b, b2, l3b, b3)
    p = _softmax(s)
    return p.reshape(_B)


# speedup vs baseline: 1.5429x; 1.5429x over previous
import functools

import jax
import jax.numpy as jnp
from jax import lax
from jax.experimental import pallas as pl
from jax.experimental.pallas import tpu as pltpu
from jax.experimental.pallas import tpu_sc as plsc

_B = 10000
_S2 = 10
_S1 = 50
_F = 3
_N = 100000
_EMB = 128
_BPAD = 10240
_G = _B // 16
_WPC = (11, 11, 10)
_GPW = (57, 57, 63)


def _sc_colsum(feat_i, neigh_i):
    mesh = plsc.VectorSubcoreMesh(core_axis_name="c", subcore_axis_name="s")
    _C = _S2 * _S1
    _GCH = 16 * _C
    _EXR = 2000
    _EXCH = _EXR * _F
    _EXT = _N // _EXR

    @functools.partial(
        pl.kernel,
        out_type=jax.ShapeDtypeStruct((_S2 * _F * _BPAD,), jnp.float32),
        mesh=mesh,
        scratch_types=[
            pltpu.VMEM((_N,), jnp.float32),
            pltpu.VMEM((_GCH,), jnp.int32),
            pltpu.VMEM((_GCH,), jnp.int32),
            pltpu.VMEM((_S2 * 1024,), jnp.float32),
            pltpu.SemaphoreType.DMA,
            pltpu.SemaphoreType.DMA,
        ],
        compiler_params=pltpu.CompilerParams(use_tc_tiling_on_sc=False,
                                             needs_layout_passes=False),
    )
    def k(feat_hbm, neigh_hbm, out_hbm, col_v, buf_a, buf_b, stage_v,
          sem_a, sem_b):
        wid = lax.axis_index("s") * 2 + lax.axis_index("c")
        iota16 = lax.iota(jnp.int32, 16)
        iota3 = iota16 * _F
        rivC = iota16 * _C

        def ex_start(buf, sem, t):
            pltpu.async_copy(feat_hbm.at[pl.ds(t * _EXCH, _EXCH)],
                             buf.at[pl.ds(0, _EXCH)], sem)

        def ex_wait(buf, sem):
            pltpu.make_async_copy(feat_hbm.at[pl.ds(0, _EXCH)],
                                  buf.at[pl.ds(0, _EXCH)], sem).wait()

        def ex_compute(buf, t, c):
            for s in range(_EXR // 16):
                col_v[pl.ds(t * _EXR + s * 16, 16)] = plsc.bitcast(
                    plsc.load_gather(buf, [iota3 + (s * 48 + c)]),
                    jnp.float32)

        def extraction(c):
            ex_start(buf_a, sem_a, 0)
            ex_start(buf_b, sem_b, 1)

            def exbody(tt, carry):
                t0 = tt * 2
                ex_wait(buf_a, sem_a)
                ex_compute(buf_a, t0, c)

                @pl.when(tt < _EXT // 2 - 1)
                def _():
                    ex_start(buf_a, sem_a, t0 + 2)

                ex_wait(buf_b, sem_b)
                ex_compute(buf_b, t0 + 1, c)

                @pl.when(tt < _EXT // 2 - 1)
                def _():
                    ex_start(buf_b, sem_b, t0 + 3)

                return carry

            lax.fori_loop(0, _EXT // 2, exbody, 0)

        def g_start(buf, sem, gidx):
            pltpu.async_copy(neigh_hbm.at[pl.ds(gidx * _GCH, _GCH)], buf, sem)

        def g_wait(buf, sem):
            pltpu.make_async_copy(neigh_hbm.at[pl.ds(0, _GCH)], buf,
                                  sem).wait()

        def g_compute(buf, g):
            def n_body(n, carry):
                acc = jnp.zeros((16,), jnp.float32)
                base = rivC + n * _S1
                for j in range(_S1):
                    iv = plsc.load_gather(buf, [base + j])
                    acc = acc + plsc.load_gather(col_v, [iv])
                stage_v[pl.ds(n * 1024 + g * 16, 16)] = acc
                return carry

            lax.fori_loop(0, _S2, n_body, 0)

        def run_col(c, lo, gpw):
            r = wid - lo
            extraction(c)
            half = (gpw + 1) // 2

            def gclamp(g):
                return jnp.minimum(r * gpw + g, _G - 1)

            g_start(buf_a, sem_a, gclamp(0))
            g_start(buf_b, sem_b, gclamp(1))

            def gbody(gg, carry):
                g0 = gg * 2
                g_wait(buf_a, sem_a)
                g_compute(buf_a, g0)

                @pl.when(gg < half - 1)
                def _():
                    g_start(buf_a, sem_a, gclamp(g0 + 2))

                g_wait(buf_b, sem_b)
                g_compute(buf_b, g0 + 1)

                @pl.when(gg < half - 1)
                def _():
                    g_start(buf_b, sem_b, gclamp(g0 + 3))

                return carry

            lax.fori_loop(0, half, gbody, 0)
            sw = gpw * 16
            for n in range(_S2):
                pltpu.sync_copy(
                    stage_v.at[pl.ds(n * 1024, sw)],
                    out_hbm.at[pl.ds((n * _F + c) * _BPAD + r * sw, sw)])

        lo = 0
        for c in range(3):
            hi = lo + _WPC[c]

            @pl.when(jnp.logical_and(wid >= lo, wid < hi))
            def _(c=c, lo=lo):
                run_col(c, lo, _GPW[c])

            lo = hi

    return k(feat_i, neigh_i)


def _bf(x):
    return x.astype(jnp.bfloat16)


def _dot(a, b):
    return jnp.dot(a, _bf(b), preferred_element_type=jnp.float32)


def _scores_softmax(m, w1b, w2b, l1b, b1v, l2b, b2, l3b, b3):
    def body(m_ref, w1_ref, w2_ref, l1_ref, b1_ref, l2_ref, b2_ref,
             l3_ref, b3_ref, o_ref):
        w1 = w1_ref[...]
        acc = jnp.zeros((_EMB, _BPAD), jnp.float32)
        for n in range(_S2):
            acc = acc + jnp.maximum(_dot(w1, m_ref[n] / 50.0), 0.0)
        h2 = jnp.maximum(_dot(w2_ref[...], acc / 10.0), 0.0)
        x1 = jnp.maximum(_dot(l1_ref[...], h2) + b1_ref[...], 0.0)
        x2 = jnp.maximum(_dot(l2_ref[...], x1) + b2_ref[...], 0.0)
        s = _dot(l3_ref[...], x2) + b3_ref[...]
        colid = lax.broadcasted_iota(jnp.int32, (1, _BPAD), 1)
        s = jnp.where(colid < _B, s, -1e30)
        e = jnp.exp(s - jnp.max(s))
        o_ref[...] = (e / jnp.sum(e))[:, :_B]

    return pl.pallas_call(
        body,
        out_shape=jax.ShapeDtypeStruct((1, _B), jnp.float32),
    )(m, w1b, w2b, l1b, b1v, l2b, b2, l3b, b3)


def kernel(nodes, features, partition_size, partition_edges, neigh2, neigh1,
           W1, W2, L1_w, L1_b, L2_w, L2_b, L3_w, L3_b):
    feat_i = lax.bitcast_convert_type(features, jnp.int32).reshape(-1)
    neigh_i = neigh1.reshape(-1).astype(jnp.int32)
    m = _sc_colsum(feat_i, neigh_i).reshape(_S2, _F, _BPAD)

    rem = (jnp.asarray(partition_size, jnp.float32)
           - jnp.float32(partition_edges.shape[0]))
    rem_b = rem.astype(jnp.bfloat16).astype(jnp.float32)
    c128 = L1_w[:, _EMB].astype(jnp.bfloat16).astype(jnp.float32)
    b1v = (L1_b + c128 * rem_b).reshape(_EMB, 1)
    w1b = W1.astype(jnp.bfloat16)
    w2b = W2.astype(jnp.bfloat16)
    l1b = L1_w[:, :_EMB].astype(jnp.bfloat16)
    l2b = L2_w.astype(jnp.bfloat16)
    l3b = L3_w.astype(jnp.bfloat16)
    b2 = L2_b.reshape(64, 1)
    b3 = L3_b.reshape(1, 1)

    p = _scores_softmax(m, w1b, w2b, l1b, b1v, l2b, b2, l3b, b3)
    return p.reshape(_B)


# speedup vs baseline: 1.5482x; 1.0035x over previous
import functools

import jax
import jax.numpy as jnp
from jax import lax
from jax.experimental import pallas as pl
from jax.experimental.pallas import tpu as pltpu
from jax.experimental.pallas import tpu_sc as plsc

_B = 10000
_S2 = 10
_S1 = 50
_F = 3
_N = 100000
_EMB = 128
_BPAD = 10240
_G = _B // 16
_WPC = (11, 11, 10)
_GPW = (57, 57, 63)


def _sc_colsum(feat_i, neigh_i):
    mesh = plsc.VectorSubcoreMesh(core_axis_name="c", subcore_axis_name="s")
    _C = _S2 * _S1
    _GCH = 16 * _C
    _EXR = 2000
    _EXCH = _EXR * _F
    _EXT = _N // _EXR

    @functools.partial(
        pl.kernel,
        out_type=jax.ShapeDtypeStruct((_S2 * _F * _BPAD,), jnp.float32),
        mesh=mesh,
        scratch_types=[
            pltpu.VMEM((_N,), jnp.float32),
            pltpu.VMEM((_GCH,), jnp.int32),
            pltpu.VMEM((_GCH,), jnp.int32),
            pltpu.VMEM((_S2 * 1024,), jnp.float32),
            pltpu.SemaphoreType.DMA,
            pltpu.SemaphoreType.DMA,
        ],
        compiler_params=pltpu.CompilerParams(use_tc_tiling_on_sc=False,
                                             needs_layout_passes=False),
    )
    def k(feat_hbm, neigh_hbm, out_hbm, col_v, buf_a, buf_b, stage_v,
          sem_a, sem_b):
        wid = lax.axis_index("s") * 2 + lax.axis_index("c")
        iota16 = lax.iota(jnp.int32, 16)
        iota3 = iota16 * _F
        rivC = iota16 * _C

        def ex_start(buf, sem, t):
            pltpu.async_copy(feat_hbm.at[pl.ds(t * _EXCH, _EXCH)],
                             buf.at[pl.ds(0, _EXCH)], sem)

        def ex_wait(buf, sem):
            pltpu.make_async_copy(feat_hbm.at[pl.ds(0, _EXCH)],
                                  buf.at[pl.ds(0, _EXCH)], sem).wait()

        def ex_compute(buf, t, c):
            for s in range(_EXR // 16):
                col_v[pl.ds(t * _EXR + s * 16, 16)] = plsc.bitcast(
                    plsc.load_gather(buf, [iota3 + (s * 48 + c)]),
                    jnp.float32)

        def extraction(c):
            ex_start(buf_a, sem_a, 0)
            ex_start(buf_b, sem_b, 1)

            def exbody(tt, carry):
                t0 = tt * 2
                ex_wait(buf_a, sem_a)
                ex_compute(buf_a, t0, c)

                @pl.when(tt < _EXT // 2 - 1)
                def _():
                    ex_start(buf_a, sem_a, t0 + 2)

                ex_wait(buf_b, sem_b)
                ex_compute(buf_b, t0 + 1, c)

                @pl.when(tt < _EXT // 2 - 1)
                def _():
                    ex_start(buf_b, sem_b, t0 + 3)

                return carry

            lax.fori_loop(0, _EXT // 2, exbody, 0)

        def g_start(buf, sem, gidx):
            pltpu.async_copy(neigh_hbm.at[pl.ds(gidx * _GCH, _GCH)], buf, sem)

        def g_wait(buf, sem):
            pltpu.make_async_copy(neigh_hbm.at[pl.ds(0, _GCH)], buf,
                                  sem).wait()

        def g_compute(buf, g):
            def n_body(n, carry):
                acc = jnp.zeros((16,), jnp.float32)
                base = rivC + n * _S1
                for j in range(_S1):
                    iv = plsc.load_gather(buf, [base + j])
                    acc = acc + plsc.load_gather(col_v, [iv])
                stage_v[pl.ds(n * 1024 + g * 16, 16)] = acc
                return carry

            lax.fori_loop(0, _S2, n_body, 0)

        def run_col(c, lo, gpw):
            r = wid - lo
            extraction(c)
            half = (gpw + 1) // 2

            def gclamp(g):
                return jnp.minimum(r * gpw + g, _G - 1)

            g_start(buf_a, sem_a, gclamp(0))
            g_start(buf_b, sem_b, gclamp(1))

            def gbody(gg, carry):
                g0 = gg * 2
                g_wait(buf_a, sem_a)
                g_compute(buf_a, g0)

                @pl.when(gg < half - 1)
                def _():
                    g_start(buf_a, sem_a, gclamp(g0 + 2))

                g_wait(buf_b, sem_b)
                g_compute(buf_b, g0 + 1)

                @pl.when(gg < half - 1)
                def _():
                    g_start(buf_b, sem_b, gclamp(g0 + 3))

                return carry

            lax.fori_loop(0, half, gbody, 0)
            sw = gpw * 16
            for n in range(_S2):
                pltpu.sync_copy(
                    stage_v.at[pl.ds(n * 1024, sw)],
                    out_hbm.at[pl.ds((n * _F + c) * _BPAD + r * sw, sw)])

        lo = 0
        for c in range(3):
            hi = lo + _WPC[c]

            @pl.when(jnp.logical_and(wid >= lo, wid < hi))
            def _(c=c, lo=lo):
                run_col(c, lo, _GPW[c])

            lo = hi

    return k(feat_i, neigh_i)


def _bf(x):
    return x.astype(jnp.bfloat16)


def _dot(a, b):
    return jnp.dot(a, _bf(b), preferred_element_type=jnp.float32)


def _scores_softmax(m, w1b, w2b, l1b, b1v, l2b, b2, l3b, b3):
    def body(m_ref, w1_ref, w2_ref, l1_ref, b1_ref, l2_ref, b2_ref,
             l3_ref, b3_ref, o_ref):
        w1 = w1_ref[...]
        acc = jnp.zeros((_EMB, _BPAD), jnp.float32)
        for n in range(_S2):
            mn = jnp.concatenate(
                [m_ref[pl.ds((n * _F + c) * _BPAD, _BPAD)].reshape(1, _BPAD)
                 for c in range(_F)], axis=0)
            acc = acc + jnp.maximum(_dot(w1, mn / 50.0), 0.0)
        h2 = jnp.maximum(_dot(w2_ref[...], acc / 10.0), 0.0)
        x1 = jnp.maximum(_dot(l1_ref[...], h2) + b1_ref[...], 0.0)
        x2 = jnp.maximum(_dot(l2_ref[...], x1) + b2_ref[...], 0.0)
        s = _dot(l3_ref[...], x2) + b3_ref[...]
        colid = lax.broadcasted_iota(jnp.int32, (1, _BPAD), 1)
        s = jnp.where(colid < _B, s, -1e30)
        e = jnp.exp(s - jnp.max(s))
        o_ref[...] = (e / jnp.sum(e))[:, :_B]

    return pl.pallas_call(
        body,
        out_shape=jax.ShapeDtypeStruct((1, _B), jnp.float32),
    )(m, w1b, w2b, l1b, b1v, l2b, b2, l3b, b3)


def kernel(nodes, features, partition_size, partition_edges, neigh2, neigh1,
           W1, W2, L1_w, L1_b, L2_w, L2_b, L3_w, L3_b):
    feat_i = lax.bitcast_convert_type(features, jnp.int32).reshape(-1)
    neigh_i = neigh1.reshape(-1).astype(jnp.int32)
    m = _sc_colsum(feat_i, neigh_i)

    rem = (jnp.asarray(partition_size, jnp.float32)
           - jnp.float32(partition_edges.shape[0]))
    rem_b = rem.astype(jnp.bfloat16).astype(jnp.float32)
    c128 = L1_w[:, _EMB].astype(jnp.bfloat16).astype(jnp.float32)
    b1v = (L1_b + c128 * rem_b).reshape(_EMB, 1)
    w1b = W1.astype(jnp.bfloat16)
    w2b = W2.astype(jnp.bfloat16)
    l1b = L1_w[:, :_EMB].astype(jnp.bfloat16)
    l2b = L2_w.astype(jnp.bfloat16)
    l3b = L3_w.astype(jnp.bfloat16)
    b2 = L2_b.reshape(64, 1)
    b3 = L3_b.reshape(1, 1)

    p = _scores_softmax(m, w1b, w2b, l1b, b1v, l2b, b2, l3b, b3)
    return p.reshape(_B)


# speedup vs baseline: 1.5579x; 1.0062x over previous
import functools

import jax
import jax.numpy as jnp
from jax import lax
from jax.experimental import pallas as pl
from jax.experimental.pallas import tpu as pltpu
from jax.experimental.pallas import tpu_sc as plsc

_B = 10000
_S2 = 10
_S1 = 50
_F = 3
_N = 100000
_EMB = 128
_BPAD = 10240
_G = _B // 16
_WPC = (11, 11, 10)
_GPW = (57, 57, 63)


def _sc_colsum(feat_i, neigh_i):
    mesh = plsc.VectorSubcoreMesh(core_axis_name="c", subcore_axis_name="s")
    _C = _S2 * _S1
    _GCH = 16 * _C
    _EXR = 2000
    _EXCH = _EXR * _F
    _EXT = _N // _EXR

    @functools.partial(
        pl.kernel,
        out_type=jax.ShapeDtypeStruct((_S2 * _F * _BPAD,), jnp.float32),
        mesh=mesh,
        scratch_types=[
            pltpu.VMEM((_N,), jnp.float32),
            pltpu.VMEM((_GCH,), jnp.int32),
            pltpu.VMEM((_GCH,), jnp.int32),
            pltpu.VMEM((_S2 * 1024,), jnp.float32),
            pltpu.SemaphoreType.DMA,
            pltpu.SemaphoreType.DMA,
        ],
        compiler_params=pltpu.CompilerParams(use_tc_tiling_on_sc=False,
                                             needs_layout_passes=False),
    )
    def k(feat_hbm, neigh_hbm, out_hbm, col_v, buf_a, buf_b, stage_v,
          sem_a, sem_b):
        wid = lax.axis_index("s") * 2 + lax.axis_index("c")
        iota16 = lax.iota(jnp.int32, 16)
        iota3 = iota16 * _F
        rivC = iota16 * _C

        def ex_start(buf, sem, t):
            pltpu.async_copy(feat_hbm.at[pl.ds(t * _EXCH, _EXCH)],
                             buf.at[pl.ds(0, _EXCH)], sem)

        def ex_wait(buf, sem):
            pltpu.make_async_copy(feat_hbm.at[pl.ds(0, _EXCH)],
                                  buf.at[pl.ds(0, _EXCH)], sem).wait()

        def ex_compute(buf, t, c):
            for s in range(_EXR // 16):
                col_v[pl.ds(t * _EXR + s * 16, 16)] = plsc.bitcast(
                    plsc.load_gather(buf, [iota3 + (s * 48 + c)]),
                    jnp.float32)

        def extraction(c):
            ex_start(buf_a, sem_a, 0)
            ex_start(buf_b, sem_b, 1)

            def exbody(tt, carry):
                t0 = tt * 2
                ex_wait(buf_a, sem_a)
                ex_compute(buf_a, t0, c)

                @pl.when(tt < _EXT // 2 - 1)
                def _():
                    ex_start(buf_a, sem_a, t0 + 2)

                ex_wait(buf_b, sem_b)
                ex_compute(buf_b, t0 + 1, c)

                @pl.when(tt < _EXT // 2 - 1)
                def _():
                    ex_start(buf_b, sem_b, t0 + 3)

                return carry

            lax.fori_loop(0, _EXT // 2, exbody, 0)

        def g_start(buf, sem, gidx):
            pltpu.async_copy(neigh_hbm.at[pl.ds(gidx * _GCH, _GCH)], buf, sem)

        def g_wait(buf, sem):
            pltpu.make_async_copy(neigh_hbm.at[pl.ds(0, _GCH)], buf,
                                  sem).wait()

        def g_compute(buf, g):
            def n_body(n, carry):
                acc = jnp.zeros((16,), jnp.float32)
                base = rivC + n * _S1
                for j in range(_S1):
                    iv = plsc.load_gather(buf, [base + j])
                    acc = acc + plsc.load_gather(col_v, [iv])
                stage_v[pl.ds(n * 1024 + g * 16, 16)] = acc
                return carry

            lax.fori_loop(0, _S2, n_body, 0)

        def run_col(c, lo, gpw):
            r = wid - lo
            extraction(c)
            half = (gpw + 1) // 2

            def gclamp(g):
                return jnp.minimum(r * gpw + g, _G - 1)

            g_start(buf_a, sem_a, gclamp(0))
            g_start(buf_b, sem_b, gclamp(1))

            def gbody(gg, carry):
                g0 = gg * 2
                g_wait(buf_a, sem_a)
                g_compute(buf_a, g0)

                @pl.when(gg < half - 1)
                def _():
                    g_start(buf_a, sem_a, gclamp(g0 + 2))

                g_wait(buf_b, sem_b)
                g_compute(buf_b, g0 + 1)

                @pl.when(gg < half - 1)
                def _():
                    g_start(buf_b, sem_b, gclamp(g0 + 3))

                return carry

            lax.fori_loop(0, half, gbody, 0)
            sw = gpw * 16
            for n in range(_S2):
                pltpu.sync_copy(
                    stage_v.at[pl.ds(n * 1024, sw)],
                    out_hbm.at[pl.ds((n * _F + c) * _BPAD + r * sw, sw)])

        lo = 0
        for c in range(3):
            hi = lo + _WPC[c]

            @pl.when(jnp.logical_and(wid >= lo, wid < hi))
            def _(c=c, lo=lo):
                run_col(c, lo, _GPW[c])

            lo = hi

    return k(feat_i, neigh_i)


def _bf(x):
    return x.astype(jnp.bfloat16)


def _dot(a, b):
    return jnp.dot(a, _bf(b), preferred_element_type=jnp.float32)


def _scores_softmax(m, w1b, w2b, l1b, b1, remv, l2b, b2, l3b, b3):
    def body(m_ref, w1_ref, w2_ref, l1_ref, b1_ref, rem_ref, l2_ref,
             b2_ref, l3_ref, b3_ref, o_ref):
        w1 = w1_ref[...]
        acc = jnp.zeros((_EMB, _BPAD), jnp.float32)
        for n in range(_S2):
            mn = jnp.concatenate(
                [m_ref[pl.ds((n * _F + c) * _BPAD, _BPAD)].reshape(1, _BPAD)
                 for c in range(_F)], axis=0)
            acc = acc + jnp.maximum(_dot(w1, mn / 50.0), 0.0)
        h2 = jnp.maximum(_dot(w2_ref[...], acc / 10.0), 0.0)
        xt = jnp.concatenate(
            [h2, jnp.broadcast_to(rem_ref[...], (1, _BPAD))], axis=0)
        x1 = jnp.maximum(_dot(l1_ref[...], xt) + b1_ref[...], 0.0)
        x2 = jnp.maximum(_dot(l2_ref[...], x1) + b2_ref[...], 0.0)
        s = _dot(l3_ref[...], x2) + b3_ref[...]
        colid = lax.broadcasted_iota(jnp.int32, (1, _BPAD), 1)
        s = jnp.where(colid < _B, s, -1e30)
        e = jnp.exp(s - jnp.max(s))
        o_ref[...] = (e / jnp.sum(e))[:, :_B]

    return pl.pallas_call(
        body,
        out_shape=jax.ShapeDtypeStruct((1, _B), jnp.float32),
    )(m, w1b, w2b, l1b, b1, remv, l2b, b2, l3b, b3)


def kernel(nodes, features, partition_size, partition_edges, neigh2, neigh1,
           W1, W2, L1_w, L1_b, L2_w, L2_b, L3_w, L3_b):
    feat_i = lax.bitcast_convert_type(features, jnp.int32).reshape(-1)
    neigh_i = neigh1.reshape(-1).astype(jnp.int32)
    m = _sc_colsum(feat_i, neigh_i)

    rem = (jnp.asarray(partition_size, jnp.float32)
           - jnp.float32(partition_edges.shape[0]))
    remv = rem.reshape(1, 1)
    b1 = L1_b.reshape(_EMB, 1)
    w1b = W1.astype(jnp.bfloat16)
    w2b = W2.astype(jnp.bfloat16)
    l1b = L1_w.astype(jnp.bfloat16)
    l2b = L2_w.astype(jnp.bfloat16)
    l3b = L3_w.astype(jnp.bfloat16)
    b2 = L2_b.reshape(64, 1)
    b3 = L3_b.reshape(1, 1)

    p = _scores_softmax(m, w1b, w2b, l1b, b1, remv, l2b, b2, l3b, b3)
    return p.reshape(_B)
